# Initial kernel scaffold; baseline (speedup 1.0000x reference)
import jax
import jax.numpy as jnp
from jax.experimental import pallas as pl


def kernel(x1, v1, v5, v6r, v7r):
    raise NotImplementedError("write your pallas kernel here")


# speedup vs baseline: 3.5339x; 3.5339x over previous
import jax
import jax.numpy as jnp
from jax.experimental import pallas as pl

_B, _N = 4, 1024
_TD = 256


def _colmax_body(v5_ref, x6_ref):
    x6_ref[0, 0, :] = jnp.max(v5_ref[0], axis=0)


def _main_body(x1_ref, v1_ref, v7r_ref, x6_ref, x10_ref, x11_ref, x12_ref):
    x1t = x1_ref[0]
    v1t = v1_ref[0]
    v7t = v7r_ref[0]
    x6v = x6_ref[0, 0]
    x9 = jax.nn.sigmoid(x1t + v7t)
    p = x9 * v1t
    top_a = p * x1t
    x6col = x6v[:, None]
    top_b = p * x6col
    x6b = jnp.broadcast_to(x6col, x1t.shape)
    x10_ref[0, :_N, :] = x1t
    x10_ref[0, _N:, :] = x6b
    x12_ref[0, :_N, :] = x1t + top_a
    x12_ref[0, _N:, :] = x6b + top_b
    x11_ref[0, :, :_N] = top_a.T
    x11_ref[0, :, _N:] = top_b.T


def kernel(x1, v1, v5, v6r, v7r):
    del v6r
    B, N, D = x1.shape

    x6 = pl.pallas_call(
        _colmax_body,
        grid=(B,),
        in_specs=[pl.BlockSpec((1, N, D), lambda b: (b, 0, 0))],
        out_specs=pl.BlockSpec((1, 1, D), lambda b: (b, 0, 0)),
        out_shape=jax.ShapeDtypeStruct((B, 1, D), jnp.float32),
    )(v5)

    strip = pl.BlockSpec((1, N, _TD), lambda b, d: (b, 0, d))
    x10, x11, x12 = pl.pallas_call(
        _main_body,
        grid=(B, D // _TD),
        in_specs=[
            strip,
            strip,
            strip,
            pl.BlockSpec((1, 1, N), lambda b, d: (b, 0, 0)),
        ],
        out_specs=[
            pl.BlockSpec((1, 2 * N, _TD), lambda b, d: (b, 0, d)),
            pl.BlockSpec((1, _TD, 2 * N), lambda b, d: (b, d, 0)),
            pl.BlockSpec((1, 2 * N, _TD), lambda b, d: (b, 0, d)),
        ],
        out_shape=[
            jax.ShapeDtypeStruct((B, 2 * N, D), jnp.float32),
            jax.ShapeDtypeStruct((B, D, 2 * N), jnp.float32),
            jax.ShapeDtypeStruct((B, 2 * N, D), jnp.float32),
        ],
    )(x1, v1, v7r, x6)
    return (x10, x11, x12)
